# Initial kernel scaffold; baseline (speedup 1.0000x reference)
import jax
import jax.numpy as jnp
from jax.experimental import pallas as pl


def kernel(logits, target):
    raise NotImplementedError("write your pallas kernel here")


# speedup vs baseline: 1.7290x; 1.7290x over previous
import functools

import jax
import jax.numpy as jnp
from jax import lax
from jax.experimental import pallas as pl
from jax.experimental.pallas import tpu as pltpu
from jax.experimental.pallas import tpu_sc as plsc

V = 100000
NROWS = 512
R = 16
NRB = NROWS // R
WS = 256
NFULL = V // WS
TAIL_VALID = V - NFULL * WS
PAD_W = (NFULL + 1) * WS
MIN = float(jnp.finfo(jnp.float32).min)

SC_NC = 2
SC_NS = 16
SC_NW = SC_NC * SC_NS
SC_RPW = NROWS // SC_NW


def _sc_gather_target(logits_flat, tgt):
    mesh = plsc.VectorSubcoreMesh(core_axis_name="c", subcore_axis_name="s")

    @functools.partial(
        pl.kernel,
        mesh=mesh,
        out_type=jax.ShapeDtypeStruct((NROWS,), jnp.float32),
        scratch_types=[
            pltpu.VMEM((SC_RPW,), jnp.int32),
            pltpu.VMEM((SC_RPW,), jnp.float32),
            pltpu.SemaphoreType.DMA,
        ],
    )
    def k(lf, tg, out, idxv, valv, sem):
        wid = lax.axis_index("s") * SC_NC + lax.axis_index("c")
        base = wid * SC_RPW
        pltpu.sync_copy(tg.at[pl.ds(base, SC_RPW)], idxv)
        rows = base + lax.iota(jnp.int32, SC_RPW)
        idxv[...] = rows * V + idxv[...]
        pltpu.async_copy(lf.at[idxv], valv, sem).wait()
        pltpu.sync_copy(valv, out.at[pl.ds(base, SC_RPW)])

    return k(logits_flat, tgt)


def _insert5(state, x):
    r1, r2, r3, r4, r5 = state
    m1 = jnp.maximum(r1, x)
    v1 = jnp.minimum(r1, x)
    m2 = jnp.maximum(r2, v1)
    v2 = jnp.minimum(r2, v1)
    m3 = jnp.maximum(r3, v2)
    v3 = jnp.minimum(r3, v2)
    m4 = jnp.maximum(r4, v3)
    v4 = jnp.minimum(r4, v3)
    m5 = jnp.maximum(r5, v4)
    return (m1, m2, m3, m4, m5)


def _make_tc_body(rows, ws, nfull, tail_valid, nrb, v):
    def body(x_ref, t_ref, out_ref, acc_ref):
        rb = pl.program_id(0)
        lane = lax.broadcasted_iota(jnp.int32, (rows, ws), 1)

        def p1(j, c):
            x = x_ref[:, pl.ds(pl.multiple_of(j * ws, ws), ws)]
            return _insert5(c, x)

        init = tuple(jnp.full((rows, ws), MIN, jnp.float32) for _ in range(5))
        st = lax.fori_loop(0, nfull, p1, init)
        xt = x_ref[:, pl.ds(nfull * ws, ws)]
        xt = jnp.where(lane < tail_valid, xt, MIN)
        r1, r2, r3, r4, r5 = _insert5(st, xt)

        def p2(j, s):
            x = x_ref[:, pl.ds(pl.multiple_of(j * ws, ws), ws)]
            return s + jnp.exp(x - r1)

        s = lax.fori_loop(0, nfull, p2, jnp.zeros((rows, ws), jnp.float32))
        xt2 = x_ref[:, pl.ds(nfull * ws, ws)]
        s = s + jnp.where(lane < tail_valid, jnp.exp(xt2 - r1), 0.0)

        m_row = jnp.max(r1, axis=1, keepdims=True)
        s_row = jnp.sum(s * jnp.exp(r1 - m_row), axis=1, keepdims=True)

        cur = [r1, r2, r3, r4, r5]
        vs = []
        for k in range(5):
            cm = functools.reduce(jnp.maximum, cur)
            vk = jnp.max(cm, axis=1, keepdims=True)
            vs.append(vk)
            if k < 4:
                cur = [jnp.where(a == vk, MIN, a) for a in cur]

        sum5 = functools.reduce(jnp.add, [jnp.exp(x - m_row) for x in vs])
        tpos = t_ref[0]
        pos = jnp.exp(tpos - m_row) / s_row
        in5 = (tpos >= vs[4]).astype(jnp.float32)
        cnt = 5.0 - in5
        sneg = sum5 / s_row - in5 * pos
        loss = -(cnt * pos - sneg) / cnt
        bsum = jnp.sum(loss)

        prev = jnp.where(rb == 0, 0.0, acc_ref[0])
        acc_ref[0] = prev + bsum

        @pl.when(rb == nrb - 1)
        def _():
            val = acc_ref[0] * jnp.float32(1.0 / (rows * nrb))
            out_ref[...] = jnp.full((1, 1), val, jnp.float32)

    return body


_tc_body = _make_tc_body(R, WS, NFULL, TAIL_VALID, NRB, V)


def _tc_loss(logits2, tvals3):
    return pl.pallas_call(
        _tc_body,
        grid=(NRB,),
        in_specs=[
            pl.BlockSpec((R, PAD_W), lambda i: (i, 0)),
            pl.BlockSpec((1, R, 1), lambda i: (i, 0, 0)),
        ],
        out_specs=pl.BlockSpec((1, 1), lambda i: (0, 0)),
        out_shape=jax.ShapeDtypeStruct((1, 1), jnp.float32),
        scratch_shapes=[pltpu.SMEM((1,), jnp.float32)],
    )(logits2, tvals3)


def kernel(logits, target):
    b, s, v = logits.shape
    logits2 = logits.reshape(b * s, v)
    tgt = target.reshape(-1).astype(jnp.int32)
    tvals = _sc_gather_target(logits.reshape(-1), tgt)
    res = _tc_loss(logits2, tvals.reshape(NRB, R, 1))
    return res[0, 0]


# speedup vs baseline: 1.9252x; 1.1135x over previous
import functools

import jax
import jax.numpy as jnp
from jax import lax
from jax.experimental import pallas as pl
from jax.experimental.pallas import tpu as pltpu
from jax.experimental.pallas import tpu_sc as plsc

V = 100000
NROWS = 512
R = 16
NRB = NROWS // R
WS = 128
NFULL = V // WS
TAIL_VALID = V - NFULL * WS
PAD_W = (NFULL + 1) * WS
NSTATES = 4
MIN = float(jnp.finfo(jnp.float32).min)

SC_NC = 2
SC_NS = 16
SC_NW = SC_NC * SC_NS
SC_RPW = NROWS // SC_NW


def _sc_gather_target(logits_flat, tgt):
    mesh = plsc.VectorSubcoreMesh(core_axis_name="c", subcore_axis_name="s")

    @functools.partial(
        pl.kernel,
        mesh=mesh,
        out_type=jax.ShapeDtypeStruct((NROWS,), jnp.float32),
        scratch_types=[
            pltpu.VMEM((SC_RPW,), jnp.int32),
            pltpu.VMEM((SC_RPW,), jnp.float32),
            pltpu.SemaphoreType.DMA,
        ],
    )
    def k(lf, tg, out, idxv, valv, sem):
        wid = lax.axis_index("s") * SC_NC + lax.axis_index("c")
        base = wid * SC_RPW
        pltpu.sync_copy(tg.at[pl.ds(base, SC_RPW)], idxv)
        rows = base + lax.iota(jnp.int32, SC_RPW)
        idxv[...] = rows * V + idxv[...]
        pltpu.async_copy(lf.at[idxv], valv, sem).wait()
        pltpu.sync_copy(valv, out.at[pl.ds(base, SC_RPW)])

    return k(logits_flat, tgt)


def _insert5(state, x):
    r1, r2, r3, r4, r5 = state
    m1 = jnp.maximum(r1, x)
    v1 = jnp.minimum(r1, x)
    m2 = jnp.maximum(r2, v1)
    v2 = jnp.minimum(r2, v1)
    m3 = jnp.maximum(r3, v2)
    v3 = jnp.minimum(r3, v2)
    m4 = jnp.maximum(r4, v3)
    v4 = jnp.minimum(r4, v3)
    m5 = jnp.maximum(r5, v4)
    return (m1, m2, m3, m4, m5)


def _make_tc_body(rows, ws, nfull, tail_valid, nrb, v, nstates=NSTATES):
    def body(x_ref, t_ref, out_ref, acc_ref):
        rb = pl.program_id(0)
        lane = lax.broadcasted_iota(jnp.int32, (rows, ws), 1)
        u = nstates
        nloop = nfull // u

        def p1(j, c):
            sts = [c[5 * i:5 * i + 5] for i in range(u)]
            base = pl.multiple_of(j * (u * ws), ws)
            for i in range(u):
                x = x_ref[:, pl.ds(base + i * ws, ws)]
                sts[i] = _insert5(sts[i], x)
            return tuple(x for st in sts for x in st)

        init = tuple(jnp.full((rows, ws), MIN, jnp.float32)
                     for _ in range(5 * u))
        st = lax.fori_loop(0, nloop, p1, init)
        sts = [list(st[5 * i:5 * i + 5]) for i in range(u)]
        rem = list(range(nloop * u, nfull))
        for i, jj in enumerate(rem):
            x = x_ref[:, pl.ds(jj * ws, ws)]
            sts[i % u] = _insert5(sts[i % u], x)
        xt = x_ref[:, pl.ds(nfull * ws, ws)]
        xt = jnp.where(lane < tail_valid, xt, MIN)
        i = len(rem) % u
        sts[i] = _insert5(sts[i], xt)

        r1 = functools.reduce(jnp.maximum, [s[0] for s in sts])

        def p2(j, accs):
            accs = list(accs)
            base = pl.multiple_of(j * (u * ws), ws)
            for i in range(u):
                x = x_ref[:, pl.ds(base + i * ws, ws)]
                accs[i] = accs[i] + jnp.exp(x - r1)
            return tuple(accs)

        zero = jnp.zeros((rows, ws), jnp.float32)
        accs = list(lax.fori_loop(0, nloop, p2, (zero,) * u))
        for i, jj in enumerate(rem):
            x = x_ref[:, pl.ds(jj * ws, ws)]
            accs[i % u] = accs[i % u] + jnp.exp(x - r1)
        xt2 = x_ref[:, pl.ds(nfull * ws, ws)]
        accs[0] = accs[0] + jnp.where(lane < tail_valid, jnp.exp(xt2 - r1), 0.0)
        s = functools.reduce(jnp.add, accs)

        m_row = jnp.max(r1, axis=1, keepdims=True)
        s_row = jnp.sum(s * jnp.exp(r1 - m_row), axis=1, keepdims=True)

        cur = [x for stt in sts for x in stt]
        vs = []
        for k in range(5):
            cm = functools.reduce(jnp.maximum, cur)
            vk = jnp.max(cm, axis=1, keepdims=True)
            vs.append(vk)
            if k < 4:
                cur = [jnp.where(a == vk, MIN, a) for a in cur]

        sum5 = functools.reduce(jnp.add, [jnp.exp(x - m_row) for x in vs])
        tpos = t_ref[0]
        pos = jnp.exp(tpos - m_row) / s_row
        in5 = (tpos >= vs[4]).astype(jnp.float32)
        cnt = 5.0 - in5
        sneg = sum5 / s_row - in5 * pos
        loss = -(cnt * pos - sneg) / cnt
        bsum = jnp.sum(loss)

        prev = jnp.where(rb == 0, 0.0, acc_ref[0])
        acc_ref[0] = prev + bsum

        @pl.when(rb == nrb - 1)
        def _():
            val = acc_ref[0] * jnp.float32(1.0 / (rows * nrb))
            out_ref[...] = jnp.full((1, 1), val, jnp.float32)

    return body


_tc_body = _make_tc_body(R, WS, NFULL, TAIL_VALID, NRB, V)


def _tc_loss(logits2, tvals3):
    return pl.pallas_call(
        _tc_body,
        grid=(NRB,),
        in_specs=[
            pl.BlockSpec((R, PAD_W), lambda i: (i, 0)),
            pl.BlockSpec((1, R, 1), lambda i: (i, 0, 0)),
        ],
        out_specs=pl.BlockSpec((1, 1), lambda i: (0, 0)),
        out_shape=jax.ShapeDtypeStruct((1, 1), jnp.float32),
        scratch_shapes=[pltpu.SMEM((1,), jnp.float32)],
    )(logits2, tvals3)


def kernel(logits, target):
    b, s, v = logits.shape
    logits2 = logits.reshape(b * s, v)
    tgt = target.reshape(-1).astype(jnp.int32)
    tvals = _sc_gather_target(logits.reshape(-1), tgt)
    res = _tc_loss(logits2, tvals.reshape(NRB, R, 1))
    return res[0, 0]
